# baseline (speedup 1.0000x reference)
import functools

import jax
import jax.numpy as jnp
import numpy as np
from jax import lax
from jax.experimental import pallas as pl
from jax.experimental.pallas import tpu as pltpu
from jax.experimental.pallas import tpu_sc as plsc

N_NODES = 10000
N_EDGES = 160000
BF = 5
H = 8
KCOLS = (BF + 1) * H

NC, NS, LANES = 2, 16, 16
NW = NC * NS
NROWS = 10240
ACCW = NROWS * H
E_PAD = 163840
EPW = E_PAD // NW
CHUNK = 128
NCHUNK = EPW // CHUNK


def _mm_body(x_ref, w_ref, o_ref):
    o_ref[...] = jnp.dot(x_ref[...], w_ref[...],
                         preferred_element_type=jnp.float32)


def _matmul(x, w, blk):
    m, k = x.shape
    n = w.shape[1]
    return pl.pallas_call(
        _mm_body,
        grid=(m // blk,),
        in_specs=[pl.BlockSpec((blk, k), lambda i: (i, 0)),
                  pl.BlockSpec((k, n), lambda i: (0, 0))],
        out_specs=pl.BlockSpec((blk, n), lambda i: (i, 0)),
        out_shape=jax.ShapeDtypeStruct((m, n), jnp.float32),
    )(x, w)


def _merge_body(acc_ref, a_ref, cb_ref, w_ref, a2_ref):
    s = jnp.sum(acc_ref[...], axis=0)
    st = jnp.transpose(s, (1, 0))
    h = jax.nn.relu(st + a_ref[:, BF * H:] + cb_ref[...])
    a2_ref[...] = jnp.dot(h, w_ref[...], preferred_element_type=jnp.float32)


def _merge_next(accs, a_prev, cb, w_next, blk):
    return pl.pallas_call(
        _merge_body,
        grid=(NROWS // blk,),
        in_specs=[pl.BlockSpec((NW, H, blk), lambda i: (0, 0, i)),
                  pl.BlockSpec((blk, KCOLS), lambda i: (i, 0)),
                  pl.BlockSpec((1, H), lambda i: (0, 0)),
                  pl.BlockSpec((H, KCOLS), lambda i: (0, 0))],
        out_specs=pl.BlockSpec((blk, KCOLS), lambda i: (i, 0)),
        out_shape=jax.ShapeDtypeStruct((NROWS, KCOLS), jnp.float32),
    )(accs, a_prev, cb, w_next)


def _final_body(acc_ref, a_ref, cb_ref, wsw_ref, wsb_ref, ts_ref, out_ref,
                sum_sc, max_sc):
    i = pl.program_id(0)
    nblk = pl.num_programs(0)
    blk = acc_ref.shape[2]
    st = jnp.transpose(jnp.sum(acc_ref[...], axis=0), (1, 0))
    h = jax.nn.relu(st + a_ref[:, BF * H:] + cb_ref[...])
    row = lax.broadcasted_iota(jnp.int32, (blk, 1), 0) + i * blk
    valid = row < N_NODES
    hz = jnp.where(valid, h, 0.0)
    hm = jnp.where(valid, h, -jnp.inf)
    aw = jax.nn.sigmoid(jnp.dot(h, wsw_ref[...],
                                preferred_element_type=jnp.float32)
                        + wsb_ref[...])
    psum = jnp.sum(jnp.where(valid, aw, 0.0) * hz, axis=0, keepdims=True)
    pmax = jnp.max(hm, axis=0, keepdims=True)

    @pl.when(i == 0)
    def _():
        sum_sc[...] = jnp.zeros_like(sum_sc)
        max_sc[...] = jnp.full_like(max_sc, -jnp.inf)

    sum_sc[...] += psum
    max_sc[...] = jnp.maximum(max_sc[...], pmax)

    @pl.when(i == nblk - 1)
    def _():
        gf = jnp.concatenate([sum_sc[...], max_sc[...]], axis=1)
        ch = 2 * H
        k2 = lax.broadcasted_iota(jnp.int32, (1, H), 1).astype(jnp.float32) * 2.0
        inv_freq = jnp.exp(-(k2 / ch) * float(np.log(10000.0)))
        t = ts_ref[0, 0]
        ang = t * inv_freq
        pos = jnp.concatenate([jnp.sin(ang), jnp.cos(ang)], axis=1)
        out_ref[...] = jnp.tanh(jax.nn.relu(gf + pos))


def _final(accs, a_prev, cb, wsw, wsb, ts, blk):
    return pl.pallas_call(
        _final_body,
        grid=(NROWS // blk,),
        in_specs=[pl.BlockSpec((NW, H, blk), lambda i: (0, 0, i)),
                  pl.BlockSpec((blk, KCOLS), lambda i: (i, 0)),
                  pl.BlockSpec((1, H), lambda i: (0, 0)),
                  pl.BlockSpec((H, 1), lambda i: (0, 0)),
                  pl.BlockSpec((1, 1), lambda i: (0, 0)),
                  pl.BlockSpec((1, 1), lambda i: (0, 0))],
        out_specs=pl.BlockSpec((1, 2 * H), lambda i: (0, 0)),
        out_shape=jax.ShapeDtypeStruct((1, 2 * H), jnp.float32),
        scratch_shapes=[pltpu.VMEM((1, H), jnp.float32),
                        pltpu.VMEM((1, H), jnp.float32)],
    )(accs, a_prev, cb, wsw, wsb, ts)


def _edge_body(a_hbm, src_hbm, dst_hbm, efb_hbm, out_hbm,
               acc, src_all, dst_a, dst_b, ef_a, ef_b, rows_a, rows_b,
               rows_t, sem_ga, sem_gb, sem_sa, sem_sb):
    wid = lax.axis_index("s") * NC + lax.axis_index("c")
    base = wid * EPW

    def issue(c, dst_v, ef_v, rows_v, sem_g, sem_s):
        cc = wid * NCHUNK + c
        pltpu.async_copy(dst_hbm.at[pl.ds(cc * CHUNK, CHUNK)], dst_v, sem_s)
        pltpu.async_copy(efb_hbm.at[pl.ds(cc * (BF + 1) * CHUNK,
                                          (BF + 1) * CHUNK)], ef_v, sem_s)
        pltpu.async_copy(a_hbm.at[src_all.at[pl.ds(c * CHUNK, CHUNK)]],
                         rows_v, sem_g)

    def wait(c, dst_v, ef_v, rows_v, sem_g, sem_s):
        cc = wid * NCHUNK + c
        pltpu.make_async_copy(dst_hbm.at[pl.ds(cc * CHUNK, CHUNK)], dst_v,
                              sem_s).wait()
        pltpu.make_async_copy(efb_hbm.at[pl.ds(cc * (BF + 1) * CHUNK,
                                               (BF + 1) * CHUNK)], ef_v,
                              sem_s).wait()
        pltpu.make_async_copy(a_hbm.at[src_all.at[pl.ds(c * CHUNK, CHUNK)]],
                              rows_v, sem_g).wait()

    cpats = [(lax.iota(jnp.int32, LANES) + 16 * j) * CHUNK for j in range(3)]

    def compute(ef_v, dst_v, rows_v, rows_t):
        def tr_body(g, _):
            for j in range(LANES):
                l = g * LANES + j
                lb = jnp.full((LANES,), l, jnp.int32)
                plsc.store_scatter(rows_t, [cpats[0] + lb],
                                   rows_v[l, pl.ds(0, LANES)])
                plsc.store_scatter(rows_t, [cpats[1] + lb],
                                   rows_v[l, pl.ds(LANES, LANES)])
                plsc.store_scatter(rows_t, [cpats[2] + lb],
                                   rows_v[l, pl.ds(2 * LANES, LANES)])
            return 0

        lax.fori_loop(0, CHUNK // LANES, tr_body, 0)

        def group_body(g, _):
            e0 = g * LANES
            dvec = dst_v[pl.ds(e0, LANES)]
            efv = [ef_v[pl.ds(b * CHUNK + e0, LANES)] for b in range(BF + 1)]
            for o in range(H):
                m = efv[0] * rows_t[pl.ds((0 * H + o) * CHUNK + e0, LANES)]
                for b in range(1, BF + 1):
                    m = m + efv[b] * rows_t[pl.ds((b * H + o) * CHUNK + e0,
                                                  LANES)]
                plsc.addupdate_scatter(acc, [dvec + o * NROWS], m)
            return 0

        lax.fori_loop(0, CHUNK // LANES, group_body, 0)

    bufs_a = (dst_a, ef_a, rows_a, sem_ga, sem_sa)
    bufs_b = (dst_b, ef_b, rows_b, sem_gb, sem_sb)

    pltpu.sync_copy(src_hbm.at[pl.ds(base, EPW)], src_all)
    issue(0, *bufs_a)
    issue(1, *bufs_b)

    zero16 = jnp.zeros((LANES,), jnp.float32)

    def zero_body(i, _):
        for j in range(16):
            acc[pl.ds((i * 16 + j) * LANES, LANES)] = zero16
        return 0

    lax.fori_loop(0, ACCW // (LANES * 16), zero_body, 0)

    NP = NCHUNK // 2

    def pair_body(p, _):
        c = 2 * p
        with jax.named_scope("wait_a"):
            wait(c, *bufs_a)
        with jax.named_scope("compute_a"):
            compute(ef_a, dst_a, rows_a, rows_t)

        @pl.when(p < NP - 1)
        def _():
            issue(c + 2, *bufs_a)

        with jax.named_scope("wait_b"):
            wait(c + 1, *bufs_b)
        with jax.named_scope("compute_b"):
            compute(ef_b, dst_b, rows_b, rows_t)

        @pl.when(p < NP - 1)
        def _():
            issue(c + 3, *bufs_b)

        return 0

    lax.fori_loop(0, NP, pair_body, 0)

    pltpu.sync_copy(acc, out_hbm.at[wid])


def _make_edge_kernel():
    mesh = plsc.VectorSubcoreMesh(core_axis_name="c", subcore_axis_name="s",
                                  num_cores=NC, num_subcores=NS)
    return pl.kernel(
        _edge_body,
        out_type=jax.ShapeDtypeStruct((NW, ACCW), jnp.float32),
        mesh=mesh,
        compiler_params=pltpu.CompilerParams(needs_layout_passes=False,
                                             use_tc_tiling_on_sc=False),
        scratch_types=[
            pltpu.VMEM((ACCW,), jnp.float32),
            pltpu.VMEM((EPW,), jnp.int32),
            pltpu.VMEM((CHUNK,), jnp.int32),
            pltpu.VMEM((CHUNK,), jnp.int32),
            pltpu.VMEM(((BF + 1) * CHUNK,), jnp.float32),
            pltpu.VMEM(((BF + 1) * CHUNK,), jnp.float32),
            pltpu.VMEM((CHUNK, KCOLS), jnp.float32),
            pltpu.VMEM((CHUNK, KCOLS), jnp.float32),
            pltpu.VMEM((KCOLS * CHUNK,), jnp.float32),
            pltpu.SemaphoreType.DMA,
            pltpu.SemaphoreType.DMA,
            pltpu.SemaphoreType.DMA,
            pltpu.SemaphoreType.DMA,
        ],
    )


def _make_waug(w, b, in_dim):
    t = w.reshape(in_dim, H, BF).transpose(0, 2, 1).reshape(in_dim, BF * H)
    return jnp.concatenate([t, b.reshape(in_dim, H)], axis=1)


def kernel(node_feats, edge_index, edge_feats, timestep, ef_w1, ef_b1,
           conv_b1, ef_w2, ef_b2, conv_b2, ws_w, ws_b):
    f32 = jnp.float32
    x = jnp.zeros((NROWS, node_feats.shape[1]), f32).at[:N_NODES].set(node_feats)
    npad = E_PAD - N_EDGES
    src = jnp.concatenate([edge_index[0], jnp.zeros((npad,), jnp.int32)])
    dst = jnp.concatenate([edge_index[1],
                           jnp.full((npad,), N_NODES, jnp.int32)])
    ef_aug = jnp.concatenate(
        [jnp.concatenate([edge_feats, jnp.zeros((npad, BF), f32)]),
         jnp.ones((E_PAD, 1), f32)], axis=1)
    efb = (ef_aug.T.reshape(BF + 1, E_PAD // CHUNK, CHUNK)
           .transpose(1, 0, 2).reshape(-1))

    w1aug = _make_waug(ef_w1, ef_b1, node_feats.shape[1])
    w2aug = _make_waug(ef_w2, ef_b2, H)
    cb1 = conv_b1.reshape(1, H)
    cb2 = conv_b2.reshape(1, H)

    edge_k = _make_edge_kernel()

    a1 = _matmul(x, w1aug, blk=2048)
    acc1 = edge_k(a1, src, dst, efb).reshape(NW, H, NROWS)
    a2 = _merge_next(acc1, a1, cb1, w2aug, blk=2048)
    acc2 = edge_k(a2, src, dst, efb).reshape(NW, H, NROWS)
    return _final(acc2, a2, cb2, ws_w, ws_b.reshape(1, 1), timestep, blk=2048)


# speedup vs baseline: 1.5122x; 1.5122x over previous
import functools

import jax
import jax.numpy as jnp
import numpy as np
from jax import lax
from jax.experimental import pallas as pl
from jax.experimental.pallas import tpu as pltpu
from jax.experimental.pallas import tpu_sc as plsc

N_NODES = 10000
N_EDGES = 160000
BF = 5
H = 8
KCOLS = (BF + 1) * H

NC, NS, LANES = 2, 16, 16
NW = NC * NS
NROWS = 10240
ACCW = NROWS * H
E_PAD = 163840
EPW = E_PAD // NW
CHUNK = 128
NCHUNK = EPW // CHUNK
TPITCH = CHUNK + 1


def _mm_body(x_ref, w_ref, o_ref):
    o_ref[...] = jnp.dot(x_ref[...], w_ref[...],
                         preferred_element_type=jnp.float32)


def _matmul(x, w, blk):
    m, k = x.shape
    n = w.shape[1]
    return pl.pallas_call(
        _mm_body,
        grid=(m // blk,),
        in_specs=[pl.BlockSpec((blk, k), lambda i: (i, 0)),
                  pl.BlockSpec((k, n), lambda i: (0, 0))],
        out_specs=pl.BlockSpec((blk, n), lambda i: (i, 0)),
        out_shape=jax.ShapeDtypeStruct((m, n), jnp.float32),
    )(x, w)


def _merge_body(acc_ref, a_ref, cb_ref, w_ref, a2_ref):
    s = jnp.sum(acc_ref[...], axis=0)
    st = jnp.transpose(s, (1, 0))
    h = jax.nn.relu(st + a_ref[:, BF * H:] + cb_ref[...])
    a2_ref[...] = jnp.dot(h, w_ref[...], preferred_element_type=jnp.float32)


def _merge_next(accs, a_prev, cb, w_next, blk):
    return pl.pallas_call(
        _merge_body,
        grid=(NROWS // blk,),
        in_specs=[pl.BlockSpec((NW, H, blk), lambda i: (0, 0, i)),
                  pl.BlockSpec((blk, KCOLS), lambda i: (i, 0)),
                  pl.BlockSpec((1, H), lambda i: (0, 0)),
                  pl.BlockSpec((H, KCOLS), lambda i: (0, 0))],
        out_specs=pl.BlockSpec((blk, KCOLS), lambda i: (i, 0)),
        out_shape=jax.ShapeDtypeStruct((NROWS, KCOLS), jnp.float32),
    )(accs, a_prev, cb, w_next)


def _final_body(acc_ref, a_ref, cb_ref, wsw_ref, wsb_ref, ts_ref, out_ref,
                sum_sc, max_sc):
    i = pl.program_id(0)
    nblk = pl.num_programs(0)
    blk = acc_ref.shape[2]
    st = jnp.transpose(jnp.sum(acc_ref[...], axis=0), (1, 0))
    h = jax.nn.relu(st + a_ref[:, BF * H:] + cb_ref[...])
    row = lax.broadcasted_iota(jnp.int32, (blk, 1), 0) + i * blk
    valid = row < N_NODES
    hz = jnp.where(valid, h, 0.0)
    hm = jnp.where(valid, h, -jnp.inf)
    aw = jax.nn.sigmoid(jnp.dot(h, wsw_ref[...],
                                preferred_element_type=jnp.float32)
                        + wsb_ref[...])
    psum = jnp.sum(jnp.where(valid, aw, 0.0) * hz, axis=0, keepdims=True)
    pmax = jnp.max(hm, axis=0, keepdims=True)

    @pl.when(i == 0)
    def _():
        sum_sc[...] = jnp.zeros_like(sum_sc)
        max_sc[...] = jnp.full_like(max_sc, -jnp.inf)

    sum_sc[...] += psum
    max_sc[...] = jnp.maximum(max_sc[...], pmax)

    @pl.when(i == nblk - 1)
    def _():
        gf = jnp.concatenate([sum_sc[...], max_sc[...]], axis=1)
        ch = 2 * H
        k2 = lax.broadcasted_iota(jnp.int32, (1, H), 1).astype(jnp.float32) * 2.0
        inv_freq = jnp.exp(-(k2 / ch) * float(np.log(10000.0)))
        t = ts_ref[0, 0]
        ang = t * inv_freq
        pos = jnp.concatenate([jnp.sin(ang), jnp.cos(ang)], axis=1)
        out_ref[...] = jnp.tanh(jax.nn.relu(gf + pos))


def _final(accs, a_prev, cb, wsw, wsb, ts, blk):
    return pl.pallas_call(
        _final_body,
        grid=(NROWS // blk,),
        in_specs=[pl.BlockSpec((NW, H, blk), lambda i: (0, 0, i)),
                  pl.BlockSpec((blk, KCOLS), lambda i: (i, 0)),
                  pl.BlockSpec((1, H), lambda i: (0, 0)),
                  pl.BlockSpec((H, 1), lambda i: (0, 0)),
                  pl.BlockSpec((1, 1), lambda i: (0, 0)),
                  pl.BlockSpec((1, 1), lambda i: (0, 0))],
        out_specs=pl.BlockSpec((1, 2 * H), lambda i: (0, 0)),
        out_shape=jax.ShapeDtypeStruct((1, 2 * H), jnp.float32),
        scratch_shapes=[pltpu.VMEM((1, H), jnp.float32),
                        pltpu.VMEM((1, H), jnp.float32)],
    )(accs, a_prev, cb, wsw, wsb, ts)


def _edge_body(a_hbm, src_hbm, dst_hbm, efb_hbm, out_hbm,
               acc, src_all, dst_a, dst_b, ef_a, ef_b, rows_a, rows_b,
               rows_t, sem_ga, sem_gb, sem_sa, sem_sb):
    wid = lax.axis_index("s") * NC + lax.axis_index("c")
    base = wid * EPW

    def issue(c, dst_v, ef_v, rows_v, sem_g, sem_s):
        cc = wid * NCHUNK + c
        pltpu.async_copy(dst_hbm.at[pl.ds(cc * CHUNK, CHUNK)], dst_v, sem_s)
        pltpu.async_copy(efb_hbm.at[pl.ds(cc * (BF + 1) * CHUNK,
                                          (BF + 1) * CHUNK)], ef_v, sem_s)
        pltpu.async_copy(a_hbm.at[src_all.at[pl.ds(c * CHUNK, CHUNK)]],
                         rows_v, sem_g)

    def wait(c, dst_v, ef_v, rows_v, sem_g, sem_s):
        cc = wid * NCHUNK + c
        pltpu.make_async_copy(dst_hbm.at[pl.ds(cc * CHUNK, CHUNK)], dst_v,
                              sem_s).wait()
        pltpu.make_async_copy(efb_hbm.at[pl.ds(cc * (BF + 1) * CHUNK,
                                               (BF + 1) * CHUNK)], ef_v,
                              sem_s).wait()
        pltpu.make_async_copy(a_hbm.at[src_all.at[pl.ds(c * CHUNK, CHUNK)]],
                              rows_v, sem_g).wait()

    cpats = [(lax.iota(jnp.int32, LANES) + 16 * j) * TPITCH for j in range(3)]

    def compute(ef_v, dst_v, rows_v, rows_t):
        def tr_body(g, _):
            for j in range(LANES):
                l = g * LANES + j
                lb = jnp.full((LANES,), l, jnp.int32)
                plsc.store_scatter(rows_t, [cpats[0] + lb],
                                   rows_v[l, pl.ds(0, LANES)])
                plsc.store_scatter(rows_t, [cpats[1] + lb],
                                   rows_v[l, pl.ds(LANES, LANES)])
                plsc.store_scatter(rows_t, [cpats[2] + lb],
                                   rows_v[l, pl.ds(2 * LANES, LANES)])
            return 0

        lax.fori_loop(0, CHUNK // LANES, tr_body, 0)

        def group_body(g, _):
            e0 = g * LANES
            dvec = dst_v[pl.ds(e0, LANES)]
            efv = [ef_v[pl.ds(b * CHUNK + e0, LANES)] for b in range(BF + 1)]
            for o in range(H):
                m = efv[0] * rows_t[pl.ds((0 * H + o) * TPITCH + e0, LANES)]
                for b in range(1, BF + 1):
                    m = m + efv[b] * rows_t[pl.ds((b * H + o) * TPITCH + e0,
                                                  LANES)]
                plsc.addupdate_scatter(acc, [dvec + o * NROWS], m)
            return 0

        lax.fori_loop(0, CHUNK // LANES, group_body, 0)

    bufs_a = (dst_a, ef_a, rows_a, sem_ga, sem_sa)
    bufs_b = (dst_b, ef_b, rows_b, sem_gb, sem_sb)

    pltpu.sync_copy(src_hbm.at[pl.ds(base, EPW)], src_all)
    issue(0, *bufs_a)
    issue(1, *bufs_b)

    zero16 = jnp.zeros((LANES,), jnp.float32)

    def zero_body(i, _):
        for j in range(16):
            acc[pl.ds((i * 16 + j) * LANES, LANES)] = zero16
        return 0

    lax.fori_loop(0, ACCW // (LANES * 16), zero_body, 0)

    NP = NCHUNK // 2

    def pair_body(p, _):
        c = 2 * p
        with jax.named_scope("wait_a"):
            wait(c, *bufs_a)
        with jax.named_scope("compute_a"):
            compute(ef_a, dst_a, rows_a, rows_t)

        @pl.when(p < NP - 1)
        def _():
            issue(c + 2, *bufs_a)

        with jax.named_scope("wait_b"):
            wait(c + 1, *bufs_b)
        with jax.named_scope("compute_b"):
            compute(ef_b, dst_b, rows_b, rows_t)

        @pl.when(p < NP - 1)
        def _():
            issue(c + 3, *bufs_b)

        return 0

    lax.fori_loop(0, NP, pair_body, 0)

    pltpu.sync_copy(acc, out_hbm.at[wid])


def _make_edge_kernel():
    mesh = plsc.VectorSubcoreMesh(core_axis_name="c", subcore_axis_name="s",
                                  num_cores=NC, num_subcores=NS)
    return pl.kernel(
        _edge_body,
        out_type=jax.ShapeDtypeStruct((NW, ACCW), jnp.float32),
        mesh=mesh,
        compiler_params=pltpu.CompilerParams(needs_layout_passes=False,
                                             use_tc_tiling_on_sc=False),
        scratch_types=[
            pltpu.VMEM((ACCW,), jnp.float32),
            pltpu.VMEM((EPW,), jnp.int32),
            pltpu.VMEM((CHUNK,), jnp.int32),
            pltpu.VMEM((CHUNK,), jnp.int32),
            pltpu.VMEM(((BF + 1) * CHUNK,), jnp.float32),
            pltpu.VMEM(((BF + 1) * CHUNK,), jnp.float32),
            pltpu.VMEM((CHUNK, KCOLS), jnp.float32),
            pltpu.VMEM((CHUNK, KCOLS), jnp.float32),
            pltpu.VMEM((KCOLS * TPITCH,), jnp.float32),
            pltpu.SemaphoreType.DMA,
            pltpu.SemaphoreType.DMA,
            pltpu.SemaphoreType.DMA,
            pltpu.SemaphoreType.DMA,
        ],
    )


def _make_waug(w, b, in_dim):
    t = w.reshape(in_dim, H, BF).transpose(0, 2, 1).reshape(in_dim, BF * H)
    return jnp.concatenate([t, b.reshape(in_dim, H)], axis=1)


def kernel(node_feats, edge_index, edge_feats, timestep, ef_w1, ef_b1,
           conv_b1, ef_w2, ef_b2, conv_b2, ws_w, ws_b):
    f32 = jnp.float32
    x = jnp.zeros((NROWS, node_feats.shape[1]), f32).at[:N_NODES].set(node_feats)
    npad = E_PAD - N_EDGES
    src = jnp.concatenate([edge_index[0], jnp.zeros((npad,), jnp.int32)])
    dst = jnp.concatenate([edge_index[1],
                           jnp.full((npad,), N_NODES, jnp.int32)])
    ef_aug = jnp.concatenate(
        [jnp.concatenate([edge_feats, jnp.zeros((npad, BF), f32)]),
         jnp.ones((E_PAD, 1), f32)], axis=1)
    efb = (ef_aug.T.reshape(BF + 1, E_PAD // CHUNK, CHUNK)
           .transpose(1, 0, 2).reshape(-1))

    w1aug = _make_waug(ef_w1, ef_b1, node_feats.shape[1])
    w2aug = _make_waug(ef_w2, ef_b2, H)
    cb1 = conv_b1.reshape(1, H)
    cb2 = conv_b2.reshape(1, H)

    edge_k = _make_edge_kernel()

    a1 = _matmul(x, w1aug, blk=2048)
    acc1 = edge_k(a1, src, dst, efb).reshape(NW, H, NROWS)
    a2 = _merge_next(acc1, a1, cb1, w2aug, blk=2048)
    acc2 = edge_k(a2, src, dst, efb).reshape(NW, H, NROWS)
    return _final(acc2, a2, cb2, ws_w, ws_b.reshape(1, 1), timestep, blk=2048)


# speedup vs baseline: 1.7295x; 1.1437x over previous
import functools

import jax
import jax.numpy as jnp
import numpy as np
from jax import lax
from jax.experimental import pallas as pl
from jax.experimental.pallas import tpu as pltpu
from jax.experimental.pallas import tpu_sc as plsc

N_NODES = 10000
N_EDGES = 160000
BF = 5
H = 8
KCOLS = (BF + 1) * H

NC, NS, LANES = 2, 16, 16
NW = NC * NS
NROWS = 10240
ACCW = NROWS * H
E_PAD = 163840
EPW = E_PAD // NW
CHUNK = 128
NCHUNK = EPW // CHUNK
TPITCH = CHUNK + 1


def _mm_body(x_ref, w_ref, o_ref):
    o_ref[...] = jnp.dot(x_ref[...], w_ref[...],
                         preferred_element_type=jnp.float32)


def _matmul(x, w, blk):
    m, k = x.shape
    n = w.shape[1]
    return pl.pallas_call(
        _mm_body,
        grid=(m // blk,),
        in_specs=[pl.BlockSpec((blk, k), lambda i: (i, 0)),
                  pl.BlockSpec((k, n), lambda i: (0, 0))],
        out_specs=pl.BlockSpec((blk, n), lambda i: (i, 0)),
        out_shape=jax.ShapeDtypeStruct((m, n), jnp.float32),
    )(x, w)


def _merge_body(acc_ref, a_ref, cb_ref, w_ref, a2_ref):
    s = jnp.sum(acc_ref[...], axis=0)
    st = jnp.transpose(s, (1, 0))
    h = jax.nn.relu(st + a_ref[:, BF * H:] + cb_ref[...])
    a2_ref[...] = jnp.dot(h, w_ref[...], preferred_element_type=jnp.float32)


def _merge_next(accs, a_prev, cb, w_next, blk):
    return pl.pallas_call(
        _merge_body,
        grid=(NROWS // blk,),
        in_specs=[pl.BlockSpec((NW, H, blk), lambda i: (0, 0, i)),
                  pl.BlockSpec((blk, KCOLS), lambda i: (i, 0)),
                  pl.BlockSpec((1, H), lambda i: (0, 0)),
                  pl.BlockSpec((H, KCOLS), lambda i: (0, 0))],
        out_specs=pl.BlockSpec((blk, KCOLS), lambda i: (i, 0)),
        out_shape=jax.ShapeDtypeStruct((NROWS, KCOLS), jnp.float32),
    )(accs, a_prev, cb, w_next)


def _final_body(acc_ref, a_ref, cb_ref, wsw_ref, wsb_ref, ts_ref, out_ref,
                sum_sc, max_sc):
    i = pl.program_id(0)
    nblk = pl.num_programs(0)
    blk = acc_ref.shape[2]
    st = jnp.transpose(jnp.sum(acc_ref[...], axis=0), (1, 0))
    h = jax.nn.relu(st + a_ref[:, BF * H:] + cb_ref[...])
    row = lax.broadcasted_iota(jnp.int32, (blk, 1), 0) + i * blk
    valid = row < N_NODES
    hz = jnp.where(valid, h, 0.0)
    hm = jnp.where(valid, h, -jnp.inf)
    aw = jax.nn.sigmoid(jnp.dot(h, wsw_ref[...],
                                preferred_element_type=jnp.float32)
                        + wsb_ref[...])
    psum = jnp.sum(jnp.where(valid, aw, 0.0) * hz, axis=0, keepdims=True)
    pmax = jnp.max(hm, axis=0, keepdims=True)

    @pl.when(i == 0)
    def _():
        sum_sc[...] = jnp.zeros_like(sum_sc)
        max_sc[...] = jnp.full_like(max_sc, -jnp.inf)

    sum_sc[...] += psum
    max_sc[...] = jnp.maximum(max_sc[...], pmax)

    @pl.when(i == nblk - 1)
    def _():
        gf = jnp.concatenate([sum_sc[...], max_sc[...]], axis=1)
        ch = 2 * H
        k2 = lax.broadcasted_iota(jnp.int32, (1, H), 1).astype(jnp.float32) * 2.0
        inv_freq = jnp.exp(-(k2 / ch) * float(np.log(10000.0)))
        t = ts_ref[0, 0]
        ang = t * inv_freq
        pos = jnp.concatenate([jnp.sin(ang), jnp.cos(ang)], axis=1)
        out_ref[...] = jnp.tanh(jax.nn.relu(gf + pos))


def _final(accs, a_prev, cb, wsw, wsb, ts, blk):
    return pl.pallas_call(
        _final_body,
        grid=(NROWS // blk,),
        in_specs=[pl.BlockSpec((NW, H, blk), lambda i: (0, 0, i)),
                  pl.BlockSpec((blk, KCOLS), lambda i: (i, 0)),
                  pl.BlockSpec((1, H), lambda i: (0, 0)),
                  pl.BlockSpec((H, 1), lambda i: (0, 0)),
                  pl.BlockSpec((1, 1), lambda i: (0, 0)),
                  pl.BlockSpec((1, 1), lambda i: (0, 0))],
        out_specs=pl.BlockSpec((1, 2 * H), lambda i: (0, 0)),
        out_shape=jax.ShapeDtypeStruct((1, 2 * H), jnp.float32),
        scratch_shapes=[pltpu.VMEM((1, H), jnp.float32),
                        pltpu.VMEM((1, H), jnp.float32)],
    )(accs, a_prev, cb, wsw, wsb, ts)


def _edge_body(a_hbm, src_hbm, dst_hbm, efb_hbm, out_hbm,
               acc, src_all, dst_a, dst_b, ef_a, ef_b, rows_a, rows_b,
               rows_t, sem_ga, sem_gb, sem_sa, sem_sb):
    wid = lax.axis_index("s") * NC + lax.axis_index("c")
    base = wid * EPW

    def issue(c, dst_v, ef_v, rows_v, sem_g, sem_s):
        cc = wid * NCHUNK + c
        pltpu.async_copy(dst_hbm.at[pl.ds(cc * CHUNK, CHUNK)], dst_v, sem_s)
        pltpu.async_copy(efb_hbm.at[pl.ds(cc * (BF + 1) * CHUNK,
                                          (BF + 1) * CHUNK)], ef_v, sem_s)
        pltpu.async_copy(a_hbm.at[src_all.at[pl.ds(c * CHUNK, CHUNK)]],
                         rows_v, sem_g)

    def wait(c, dst_v, ef_v, rows_v, sem_g, sem_s):
        cc = wid * NCHUNK + c
        pltpu.make_async_copy(dst_hbm.at[pl.ds(cc * CHUNK, CHUNK)], dst_v,
                              sem_s).wait()
        pltpu.make_async_copy(efb_hbm.at[pl.ds(cc * (BF + 1) * CHUNK,
                                               (BF + 1) * CHUNK)], ef_v,
                              sem_s).wait()
        pltpu.make_async_copy(a_hbm.at[src_all.at[pl.ds(c * CHUNK, CHUNK)]],
                              rows_v, sem_g).wait()

    cpats = [(lax.iota(jnp.int32, LANES) + 16 * j) * TPITCH for j in range(3)]

    def compute(ef_v, dst_v, rows_v, rows_t):
        def tr_body(g, _):
            for j in range(LANES):
                l = g * LANES + j
                lb = jnp.full((LANES,), l, jnp.int32)
                plsc.store_scatter(rows_t, [cpats[0] + lb],
                                   rows_v[l, pl.ds(0, LANES)])
                plsc.store_scatter(rows_t, [cpats[1] + lb],
                                   rows_v[l, pl.ds(LANES, LANES)])
                plsc.store_scatter(rows_t, [cpats[2] + lb],
                                   rows_v[l, pl.ds(2 * LANES, LANES)])
            return 0

        lax.fori_loop(0, CHUNK // LANES, tr_body, 0)

        def group_body(g, _):
            e0 = g * LANES
            dvec = dst_v[pl.ds(e0, LANES)]
            efv = [ef_v[pl.ds(b * CHUNK + e0, LANES)] for b in range(BF + 1)]
            for o in range(H):
                m = efv[0] * rows_t[pl.ds((0 * H + o) * TPITCH + e0, LANES)]
                for b in range(1, BF + 1):
                    m = m + efv[b] * rows_t[pl.ds((b * H + o) * TPITCH + e0,
                                                  LANES)]
                plsc.addupdate_scatter(acc, [dvec + o * NROWS], m)
            return 0

        lax.fori_loop(0, CHUNK // LANES, group_body, 0)

    bufs_a = (dst_a, ef_a, rows_a, sem_ga, sem_sa)
    bufs_b = (dst_b, ef_b, rows_b, sem_gb, sem_sb)

    pltpu.sync_copy(src_hbm.at[pl.ds(base, EPW)], src_all)
    issue(0, *bufs_a)
    issue(1, *bufs_b)

    zero16 = jnp.zeros((LANES,), jnp.float32)

    def zero_body(i, _):
        for j in range(16):
            acc[pl.ds((i * 16 + j) * LANES, LANES)] = zero16
        return 0

    lax.fori_loop(0, ACCW // (LANES * 16), zero_body, 0)

    NP = NCHUNK // 2

    def pair_body(p, _):
        c = 2 * p
        with jax.named_scope("wait_a"):
            wait(c, *bufs_a)
        with jax.named_scope("compute_a"):
            compute(ef_a, dst_a, rows_a, rows_t)

        @pl.when(p < NP - 1)
        def _():
            issue(c + 2, *bufs_a)

        with jax.named_scope("wait_b"):
            wait(c + 1, *bufs_b)
        with jax.named_scope("compute_b"):
            compute(ef_b, dst_b, rows_b, rows_t)

        @pl.when(p < NP - 1)
        def _():
            issue(c + 3, *bufs_b)

        return 0

    lax.fori_loop(0, NP, pair_body, 0)

    for o in range(H):
        pltpu.sync_copy(acc.at[pl.ds(o * NROWS, NROWS)], out_hbm.at[wid, o])


def _make_edge_kernel():
    mesh = plsc.VectorSubcoreMesh(core_axis_name="c", subcore_axis_name="s",
                                  num_cores=NC, num_subcores=NS)
    return pl.kernel(
        _edge_body,
        out_type=jax.ShapeDtypeStruct((NW, H, NROWS), jnp.float32),
        mesh=mesh,
        compiler_params=pltpu.CompilerParams(needs_layout_passes=False,
                                             use_tc_tiling_on_sc=False),
        scratch_types=[
            pltpu.VMEM((ACCW,), jnp.float32),
            pltpu.VMEM((EPW,), jnp.int32),
            pltpu.VMEM((CHUNK,), jnp.int32),
            pltpu.VMEM((CHUNK,), jnp.int32),
            pltpu.VMEM(((BF + 1) * CHUNK,), jnp.float32),
            pltpu.VMEM(((BF + 1) * CHUNK,), jnp.float32),
            pltpu.VMEM((CHUNK, KCOLS), jnp.float32),
            pltpu.VMEM((CHUNK, KCOLS), jnp.float32),
            pltpu.VMEM((KCOLS * TPITCH,), jnp.float32),
            pltpu.SemaphoreType.DMA,
            pltpu.SemaphoreType.DMA,
            pltpu.SemaphoreType.DMA,
            pltpu.SemaphoreType.DMA,
        ],
    )


def _make_waug(w, b, in_dim):
    t = w.reshape(in_dim, H, BF).transpose(0, 2, 1).reshape(in_dim, BF * H)
    return jnp.concatenate([t, b.reshape(in_dim, H)], axis=1)


def kernel(node_feats, edge_index, edge_feats, timestep, ef_w1, ef_b1,
           conv_b1, ef_w2, ef_b2, conv_b2, ws_w, ws_b):
    f32 = jnp.float32
    x = jnp.zeros((NROWS, node_feats.shape[1]), f32).at[:N_NODES].set(node_feats)
    npad = E_PAD - N_EDGES
    src = jnp.concatenate([edge_index[0], jnp.zeros((npad,), jnp.int32)])
    pad_dst = N_NODES + (jnp.arange(npad, dtype=jnp.int32)
                         % (NROWS - N_NODES))
    dst = jnp.concatenate([edge_index[1], pad_dst])
    ef_aug = jnp.concatenate(
        [jnp.concatenate([edge_feats, jnp.zeros((npad, BF), f32)]),
         jnp.ones((E_PAD, 1), f32)], axis=1)
    efb = (ef_aug.T.reshape(BF + 1, E_PAD // CHUNK, CHUNK)
           .transpose(1, 0, 2).reshape(-1))

    w1aug = _make_waug(ef_w1, ef_b1, node_feats.shape[1])
    w2aug = _make_waug(ef_w2, ef_b2, H)
    cb1 = conv_b1.reshape(1, H)
    cb2 = conv_b2.reshape(1, H)

    edge_k = _make_edge_kernel()

    a1 = _matmul(x, w1aug, blk=2048)
    acc1 = edge_k(a1, src, dst, efb)
    a2 = _merge_next(acc1, a1, cb1, w2aug, blk=2048)
    acc2 = edge_k(a2, src, dst, efb)
    return _final(acc2, a2, cb2, ws_w, ws_b.reshape(1, 1), timestep, blk=2048)


# speedup vs baseline: 1.7404x; 1.0063x over previous
import functools

import jax
import jax.numpy as jnp
import numpy as np
from jax import lax
from jax.experimental import pallas as pl
from jax.experimental.pallas import tpu as pltpu
from jax.experimental.pallas import tpu_sc as plsc

N_NODES = 10000
N_EDGES = 160000
BF = 5
H = 8
KCOLS = (BF + 1) * H

NC, NS, LANES = 2, 16, 16
NW = NC * NS
NROWS = 10240
ACCW = NROWS * H
E_PAD = 163840
EPW = E_PAD // NW
CHUNK = 128
NCHUNK = EPW // CHUNK
TPITCH = CHUNK + 1


def _mm_body(x_ref, w_ref, o_ref):
    o_ref[...] = jnp.dot(x_ref[...], w_ref[...],
                         preferred_element_type=jnp.float32)


def _matmul(x, w, blk):
    m, k = x.shape
    n = w.shape[1]
    return pl.pallas_call(
        _mm_body,
        grid=(m // blk,),
        in_specs=[pl.BlockSpec((blk, k), lambda i: (i, 0)),
                  pl.BlockSpec((k, n), lambda i: (0, 0))],
        out_specs=pl.BlockSpec((blk, n), lambda i: (i, 0)),
        out_shape=jax.ShapeDtypeStruct((m, n), jnp.float32),
    )(x, w)


def _merge_body(acc_ref, a_ref, cb_ref, w_ref, a2_ref):
    s = jnp.sum(acc_ref[...], axis=0)
    st = jnp.transpose(s, (1, 0))
    h = jax.nn.relu(st + a_ref[:, BF * H:] + cb_ref[...])
    a2_ref[...] = jnp.dot(h, w_ref[...], preferred_element_type=jnp.float32)


def _merge_next(accs, a_prev, cb, w_next, blk):
    return pl.pallas_call(
        _merge_body,
        grid=(NROWS // blk,),
        in_specs=[pl.BlockSpec((NW, H, blk), lambda i: (0, 0, i)),
                  pl.BlockSpec((blk, KCOLS), lambda i: (i, 0)),
                  pl.BlockSpec((1, H), lambda i: (0, 0)),
                  pl.BlockSpec((H, KCOLS), lambda i: (0, 0))],
        out_specs=pl.BlockSpec((blk, KCOLS), lambda i: (i, 0)),
        out_shape=jax.ShapeDtypeStruct((NROWS, KCOLS), jnp.float32),
    )(accs, a_prev, cb, w_next)


def _final_body(acc_ref, a_ref, cb_ref, wsw_ref, wsb_ref, ts_ref, out_ref,
                sum_sc, max_sc):
    i = pl.program_id(0)
    nblk = pl.num_programs(0)
    blk = acc_ref.shape[2]
    st = jnp.transpose(jnp.sum(acc_ref[...], axis=0), (1, 0))
    h = jax.nn.relu(st + a_ref[:, BF * H:] + cb_ref[...])
    row = lax.broadcasted_iota(jnp.int32, (blk, 1), 0) + i * blk
    valid = row < N_NODES
    hz = jnp.where(valid, h, 0.0)
    hm = jnp.where(valid, h, -jnp.inf)
    aw = jax.nn.sigmoid(jnp.dot(h, wsw_ref[...],
                                preferred_element_type=jnp.float32)
                        + wsb_ref[...])
    psum = jnp.sum(jnp.where(valid, aw, 0.0) * hz, axis=0, keepdims=True)
    pmax = jnp.max(hm, axis=0, keepdims=True)

    @pl.when(i == 0)
    def _():
        sum_sc[...] = jnp.zeros_like(sum_sc)
        max_sc[...] = jnp.full_like(max_sc, -jnp.inf)

    sum_sc[...] += psum
    max_sc[...] = jnp.maximum(max_sc[...], pmax)

    @pl.when(i == nblk - 1)
    def _():
        gf = jnp.concatenate([sum_sc[...], max_sc[...]], axis=1)
        ch = 2 * H
        k2 = lax.broadcasted_iota(jnp.int32, (1, H), 1).astype(jnp.float32) * 2.0
        inv_freq = jnp.exp(-(k2 / ch) * float(np.log(10000.0)))
        t = ts_ref[0, 0]
        ang = t * inv_freq
        pos = jnp.concatenate([jnp.sin(ang), jnp.cos(ang)], axis=1)
        out_ref[...] = jnp.tanh(jax.nn.relu(gf + pos))


def _final(accs, a_prev, cb, wsw, wsb, ts, blk):
    return pl.pallas_call(
        _final_body,
        grid=(NROWS // blk,),
        in_specs=[pl.BlockSpec((NW, H, blk), lambda i: (0, 0, i)),
                  pl.BlockSpec((blk, KCOLS), lambda i: (i, 0)),
                  pl.BlockSpec((1, H), lambda i: (0, 0)),
                  pl.BlockSpec((H, 1), lambda i: (0, 0)),
                  pl.BlockSpec((1, 1), lambda i: (0, 0)),
                  pl.BlockSpec((1, 1), lambda i: (0, 0))],
        out_specs=pl.BlockSpec((1, 2 * H), lambda i: (0, 0)),
        out_shape=jax.ShapeDtypeStruct((1, 2 * H), jnp.float32),
        scratch_shapes=[pltpu.VMEM((1, H), jnp.float32),
                        pltpu.VMEM((1, H), jnp.float32)],
    )(accs, a_prev, cb, wsw, wsb, ts)


def _edge_body(a_hbm, src_hbm, dst_hbm, efb_hbm, out_hbm,
               acc, src_all, dst_a, dst_b, ef_a, ef_b, rows_a, rows_b,
               rows_t, sem_ga, sem_gb, sem_sa, sem_sb):
    wid = lax.axis_index("s") * NC + lax.axis_index("c")
    base = wid * EPW

    def issue(c, dst_v, ef_v, rows_v, sem_g, sem_s):
        cc = wid * NCHUNK + c
        pltpu.async_copy(dst_hbm.at[pl.ds(cc * CHUNK, CHUNK)], dst_v, sem_s)
        pltpu.async_copy(efb_hbm.at[pl.ds(cc * (BF + 1) * CHUNK,
                                          (BF + 1) * CHUNK)], ef_v, sem_s)
        pltpu.async_copy(a_hbm.at[src_all.at[pl.ds(c * CHUNK, CHUNK)]],
                         rows_v, sem_g)

    def wait(c, dst_v, ef_v, rows_v, sem_g, sem_s):
        cc = wid * NCHUNK + c
        pltpu.make_async_copy(dst_hbm.at[pl.ds(cc * CHUNK, CHUNK)], dst_v,
                              sem_s).wait()
        pltpu.make_async_copy(efb_hbm.at[pl.ds(cc * (BF + 1) * CHUNK,
                                               (BF + 1) * CHUNK)], ef_v,
                              sem_s).wait()
        pltpu.make_async_copy(a_hbm.at[src_all.at[pl.ds(c * CHUNK, CHUNK)]],
                              rows_v, sem_g).wait()

    cpats = [(lax.iota(jnp.int32, LANES) + 16 * j) * TPITCH for j in range(3)]

    def compute(ef_v, dst_v, rows_v, rows_t):
        def group_body(g, _):
            e0 = g * LANES
            for j in range(LANES):
                l = e0 + j
                lb = jnp.full((LANES,), l, jnp.int32)
                plsc.store_scatter(rows_t, [cpats[0] + lb],
                                   rows_v[l, pl.ds(0, LANES)])
                plsc.store_scatter(rows_t, [cpats[1] + lb],
                                   rows_v[l, pl.ds(LANES, LANES)])
                plsc.store_scatter(rows_t, [cpats[2] + lb],
                                   rows_v[l, pl.ds(2 * LANES, LANES)])
            dvec = dst_v[pl.ds(e0, LANES)]
            efv = [ef_v[pl.ds(b * CHUNK + e0, LANES)] for b in range(BF + 1)]
            for o in range(H):
                m = efv[0] * rows_t[pl.ds((0 * H + o) * TPITCH + e0, LANES)]
                for b in range(1, BF + 1):
                    m = m + efv[b] * rows_t[pl.ds((b * H + o) * TPITCH + e0,
                                                  LANES)]
                plsc.addupdate_scatter(acc, [dvec + o * NROWS], m)
            return 0

        lax.fori_loop(0, CHUNK // LANES, group_body, 0)

    bufs_a = (dst_a, ef_a, rows_a, sem_ga, sem_sa)
    bufs_b = (dst_b, ef_b, rows_b, sem_gb, sem_sb)

    pltpu.sync_copy(src_hbm.at[pl.ds(base, EPW)], src_all)
    issue(0, *bufs_a)
    issue(1, *bufs_b)

    zero16 = jnp.zeros((LANES,), jnp.float32)

    def zero_body(i, _):
        for j in range(16):
            acc[pl.ds((i * 16 + j) * LANES, LANES)] = zero16
        return 0

    lax.fori_loop(0, ACCW // (LANES * 16), zero_body, 0)

    NP = NCHUNK // 2

    def pair_body(p, _):
        c = 2 * p
        with jax.named_scope("wait_a"):
            wait(c, *bufs_a)
        with jax.named_scope("compute_a"):
            compute(ef_a, dst_a, rows_a, rows_t)

        @pl.when(p < NP - 1)
        def _():
            issue(c + 2, *bufs_a)

        with jax.named_scope("wait_b"):
            wait(c + 1, *bufs_b)
        with jax.named_scope("compute_b"):
            compute(ef_b, dst_b, rows_b, rows_t)

        @pl.when(p < NP - 1)
        def _():
            issue(c + 3, *bufs_b)

        return 0

    lax.fori_loop(0, NP, pair_body, 0)

    for o in range(H):
        pltpu.sync_copy(acc.at[pl.ds(o * NROWS, NROWS)], out_hbm.at[wid, o])


def _make_edge_kernel():
    mesh = plsc.VectorSubcoreMesh(core_axis_name="c", subcore_axis_name="s",
                                  num_cores=NC, num_subcores=NS)
    return pl.kernel(
        _edge_body,
        out_type=jax.ShapeDtypeStruct((NW, H, NROWS), jnp.float32),
        mesh=mesh,
        compiler_params=pltpu.CompilerParams(needs_layout_passes=False,
                                             use_tc_tiling_on_sc=False),
        scratch_types=[
            pltpu.VMEM((ACCW,), jnp.float32),
            pltpu.VMEM((EPW,), jnp.int32),
            pltpu.VMEM((CHUNK,), jnp.int32),
            pltpu.VMEM((CHUNK,), jnp.int32),
            pltpu.VMEM(((BF + 1) * CHUNK,), jnp.float32),
            pltpu.VMEM(((BF + 1) * CHUNK,), jnp.float32),
            pltpu.VMEM((CHUNK, KCOLS), jnp.float32),
            pltpu.VMEM((CHUNK, KCOLS), jnp.float32),
            pltpu.VMEM((KCOLS * TPITCH,), jnp.float32),
            pltpu.SemaphoreType.DMA,
            pltpu.SemaphoreType.DMA,
            pltpu.SemaphoreType.DMA,
            pltpu.SemaphoreType.DMA,
        ],
    )


def _make_waug(w, b, in_dim):
    t = w.reshape(in_dim, H, BF).transpose(0, 2, 1).reshape(in_dim, BF * H)
    return jnp.concatenate([t, b.reshape(in_dim, H)], axis=1)


def kernel(node_feats, edge_index, edge_feats, timestep, ef_w1, ef_b1,
           conv_b1, ef_w2, ef_b2, conv_b2, ws_w, ws_b):
    f32 = jnp.float32
    x = jnp.zeros((NROWS, node_feats.shape[1]), f32).at[:N_NODES].set(node_feats)
    npad = E_PAD - N_EDGES
    src = jnp.concatenate([edge_index[0], jnp.zeros((npad,), jnp.int32)])
    pad_dst = N_NODES + (jnp.arange(npad, dtype=jnp.int32)
                         % (NROWS - N_NODES))
    dst = jnp.concatenate([edge_index[1], pad_dst])
    ef_aug = jnp.concatenate(
        [jnp.concatenate([edge_feats, jnp.zeros((npad, BF), f32)]),
         jnp.ones((E_PAD, 1), f32)], axis=1)
    efb = (ef_aug.T.reshape(BF + 1, E_PAD // CHUNK, CHUNK)
           .transpose(1, 0, 2).reshape(-1))

    w1aug = _make_waug(ef_w1, ef_b1, node_feats.shape[1])
    w2aug = _make_waug(ef_w2, ef_b2, H)
    cb1 = conv_b1.reshape(1, H)
    cb2 = conv_b2.reshape(1, H)

    edge_k = _make_edge_kernel()

    a1 = _matmul(x, w1aug, blk=2048)
    acc1 = edge_k(a1, src, dst, efb)
    a2 = _merge_next(acc1, a1, cb1, w2aug, blk=2048)
    acc2 = edge_k(a2, src, dst, efb)
    return _final(acc2, a2, cb2, ws_w, ws_b.reshape(1, 1), timestep, blk=2048)
